# baseline (speedup 1.0000x reference)
import functools

import jax
import jax.numpy as jnp
from jax import lax
from jax.experimental import pallas as pl
from jax.experimental.pallas import tpu as pltpu
from jax.experimental.pallas import tpu_sc as plsc

N = 10000
E = 320000
H = 128
DE = 16
L = 3
G = 128
NC = 10

NUM_CORES = 2
NUM_SUBCORES = 16
NW = NUM_CORES * NUM_SUBCORES
CHUNK = 128
CH_PER_W = 80
E_PAD = NW * CHUNK * CH_PER_W
NPAD = 10112
ROWS_PER_TILE = NPAD // NUM_SUBCORES

NT = 1000
NGRID = N // NT
ET = 1024
EGRID = E_PAD // ET

NIDX = 2
NEBUF = 2


def _edge_body(h_hbm, e_hbm, sdm_hbm, zeros_hbm, out_hbm,
               idx_v, rows_v, e_v, aggr_sh, sem_g):
    cid = lax.axis_index("c")
    sid = lax.axis_index("s")
    w = cid * NUM_SUBCORES + sid
    r0 = sid * ROWS_PER_TILE
    pltpu.sync_copy(zeros_hbm.at[pl.ds(r0, ROWS_PER_TILE)],
                    aggr_sh.at[pl.ds(r0, ROWS_PER_TILE)])
    plsc.subcore_barrier()

    base = w * CH_PER_W

    def chunk_body(k, carry):
        pltpu.sync_copy(sdm_hbm.at[pl.ds(base + k, 1)], idx_v)
        pltpu.async_copy(h_hbm.at[idx_v.at[0, 0]], rows_v, sem_g)
        pltpu.sync_copy(e_hbm.at[pl.ds((base + k) * CHUNK, CHUNK)], e_v)
        pltpu.make_async_copy(h_hbm.at[pl.ds(0, CHUNK)], rows_v,
                              sem_g).wait()

        def vbody(i, cc):
            for j in range(H // 16):
                s = (rows_v[i, pl.ds(j * 16, 16)]
                     + e_v[i, pl.ds(j * 16, 16)])
                rows_v[i, pl.ds(j * 16, 16)] = jnp.maximum(s, 0.0)
            return cc

        lax.fori_loop(0, CHUNK, vbody, 0)

        pltpu.sync_copy(rows_v, aggr_sh.at[idx_v.at[0, 1]], add=True)
        return carry

    lax.fori_loop(0, CH_PER_W, chunk_body, 0)
    plsc.subcore_barrier()
    pltpu.sync_copy(aggr_sh.at[pl.ds(r0, ROWS_PER_TILE)],
                    out_hbm.at[cid, pl.ds(r0, ROWS_PER_TILE)])


_edge_call = functools.partial(
    pl.kernel,
    out_type=jax.ShapeDtypeStruct((NUM_CORES, NPAD, H), jnp.float32),
    mesh=plsc.VectorSubcoreMesh(core_axis_name="c", subcore_axis_name="s",
                                num_cores=NUM_CORES, num_subcores=NUM_SUBCORES),
    scratch_types=[
        pltpu.VMEM((1, 2, CHUNK), jnp.int32),
        pltpu.VMEM((CHUNK, H), jnp.float32),
        pltpu.VMEM((CHUNK, H), jnp.float32),
        pltpu.VMEM_SHARED((NPAD, H), jnp.float32),
        pltpu.SemaphoreType.DMA,
    ],
)(_edge_body)


def _e_mm_body(ea_ref, w_ref, b_ref, o_ref):
    o_ref[...] = (jnp.dot(ea_ref[...], w_ref[...],
                          preferred_element_type=jnp.float32) + b_ref[...])


_e_mm = pl.pallas_call(
    _e_mm_body,
    grid=(EGRID,),
    in_specs=[
        pl.BlockSpec((ET, DE), lambda i: (i, 0)),
        pl.BlockSpec((DE, H), lambda i: (0, 0)),
        pl.BlockSpec((1, H), lambda i: (0, 0)),
    ],
    out_specs=pl.BlockSpec((ET, H), lambda i: (i, 0)),
    out_shape=jax.ShapeDtypeStruct((E_PAD, H), jnp.float32),
)


def _pre_body(h_ref, bf_ref, vn_ref, hin_ref, vt_ref):
    i = pl.program_id(0)
    iota = lax.broadcasted_iota(jnp.int32, (NT, G), 1)
    oh = jnp.where(bf_ref[...] == iota, 1.0, 0.0)
    hin = h_ref[...] + jnp.dot(oh, vn_ref[...], preferred_element_type=jnp.float32)
    hin_ref[...] = hin
    contrib = lax.dot_general(oh, hin, (((0,), (0,)), ((), ())),
                              preferred_element_type=jnp.float32)

    @pl.when(i == 0)
    def _():
        vt_ref[...] = jnp.zeros_like(vt_ref)

    vt_ref[...] += contrib


_pre = pl.pallas_call(
    _pre_body,
    grid=(NGRID,),
    in_specs=[
        pl.BlockSpec((NT, H), lambda i: (i, 0)),
        pl.BlockSpec((NT, 1), lambda i: (i, 0)),
        pl.BlockSpec((G, H), lambda i: (0, 0)),
    ],
    out_specs=[
        pl.BlockSpec((NT, H), lambda i: (i, 0)),
        pl.BlockSpec((G, H), lambda i: (0, 0)),
    ],
    out_shape=[
        jax.ShapeDtypeStruct((N, H), jnp.float32),
        jax.ShapeDtypeStruct((G, H), jnp.float32),
    ],
)


def _mlp_body(aggr_ref, hin_ref, w1_ref, b1_ref, w2_ref, b2_ref, o_ref):
    xx = aggr_ref[0] + aggr_ref[1] + hin_ref[...]
    z = jnp.dot(xx, w1_ref[...], preferred_element_type=jnp.float32) + b1_ref[...]
    z = jnp.maximum(z, 0.0)
    o_ref[...] = (jnp.dot(z, w2_ref[...], preferred_element_type=jnp.float32)
                  + b2_ref[...])


_mlp = pl.pallas_call(
    _mlp_body,
    grid=(NGRID,),
    in_specs=[
        pl.BlockSpec((NUM_CORES, NT, H), lambda i: (0, i, 0)),
        pl.BlockSpec((NT, H), lambda i: (i, 0)),
        pl.BlockSpec((H, 2 * H), lambda i: (0, 0)),
        pl.BlockSpec((1, 2 * H), lambda i: (0, 0)),
        pl.BlockSpec((2 * H, H), lambda i: (0, 0)),
        pl.BlockSpec((1, H), lambda i: (0, 0)),
    ],
    out_specs=pl.BlockSpec((NT, H), lambda i: (i, 0)),
    out_shape=jax.ShapeDtypeStruct((N, H), jnp.float32),
)


def _vn_body(vt_ref, vn_ref, v1_ref, c1_ref, a1_ref, v2_ref, c2_ref, a2_ref,
             o_ref):
    u = jnp.dot(vt_ref[...] + vn_ref[...], v1_ref[...],
                preferred_element_type=jnp.float32) + c1_ref[...]
    a1 = a1_ref[0, 0]
    u = jnp.where(u >= 0.0, u, a1 * u)
    u = jnp.dot(u, v2_ref[...], preferred_element_type=jnp.float32) + c2_ref[...]
    a2 = a2_ref[0, 0]
    o_ref[...] = jnp.where(u >= 0.0, u, a2 * u)


_vn_mlp = pl.pallas_call(
    _vn_body,
    out_shape=jax.ShapeDtypeStruct((G, H), jnp.float32),
)


def _final_body(h_ref, bf_ref, wg_ref, bg_ref, o_ref, num_acc, cnt_acc):
    i = pl.program_id(0)
    iota = lax.broadcasted_iota(jnp.int32, (NT, G), 1)
    oh = jnp.where(bf_ref[...] == iota, 1.0, 0.0)

    @pl.when(i == 0)
    def _():
        num_acc[...] = jnp.zeros_like(num_acc)
        cnt_acc[...] = jnp.zeros_like(cnt_acc)

    num_acc[...] += lax.dot_general(oh, h_ref[...], (((0,), (0,)), ((), ())),
                                    preferred_element_type=jnp.float32)
    ones = jnp.ones((NT, H), jnp.float32)
    cnt_acc[...] += lax.dot_general(oh, ones, (((0,), (0,)), ((), ())),
                                    preferred_element_type=jnp.float32)

    @pl.when(i == NGRID - 1)
    def _():
        hg = num_acc[...] / jnp.maximum(cnt_acc[...], 1.0)
        o_ref[...] = (jnp.dot(hg, wg_ref[...],
                              preferred_element_type=jnp.float32) + bg_ref[...])


_final = pl.pallas_call(
    _final_body,
    grid=(NGRID,),
    in_specs=[
        pl.BlockSpec((NT, H), lambda i: (i, 0)),
        pl.BlockSpec((NT, 1), lambda i: (i, 0)),
        pl.BlockSpec((H, H), lambda i: (0, 0)),
        pl.BlockSpec((1, H), lambda i: (0, 0)),
    ],
    out_specs=pl.BlockSpec((G, H), lambda i: (0, 0)),
    out_shape=jax.ShapeDtypeStruct((G, H), jnp.float32),
    scratch_shapes=[
        pltpu.VMEM((G, H), jnp.float32),
        pltpu.VMEM((G, H), jnp.float32),
    ],
)


def kernel(x, edge_index, edge_attr, batch, node_emb, vn_emb, We, be, W1, b1,
           g1, bt1, W2, b2, V1, c1, vg1, vb1, pa1, V2, c2, vg2, vb2, pa2,
           Wg, bg):
    s = 1.0 / jnp.sqrt(jnp.float32(1.0 + 1e-5))
    W1f = W1 * (g1 * s)[:, None, :]
    b1f = b1 * (g1 * s) + bt1
    V1f = V1 * (vg1 * s)[:, None, :]
    c1f = c1 * (vg1 * s) + vb1
    V2f = V2 * (vg2 * s)[:, None, :]
    c2f = c2 * (vg2 * s) + vb2

    h = jnp.take(node_emb, x, axis=0)
    vn = jnp.take(vn_emb, jnp.zeros((G,), jnp.int32), axis=0)
    batchf = batch.reshape(N, 1)

    pad = E_PAD - E
    srcp = jnp.concatenate([edge_index[0], jnp.zeros((pad,), jnp.int32)])
    dstp = jnp.concatenate([edge_index[1], jnp.full((pad,), N, jnp.int32)])
    sdm = jnp.stack([srcp.reshape(E_PAD // CHUNK, CHUNK),
                     dstp.reshape(E_PAD // CHUNK, CHUNK)], axis=1)
    eap = jnp.concatenate([edge_attr, jnp.zeros((pad, DE), jnp.float32)])
    zeros_buf = jnp.zeros((NPAD, H), jnp.float32)

    wg_pad = jnp.zeros((H, H), jnp.float32).at[:, :NC].set(Wg)
    bg_pad = jnp.zeros((1, H), jnp.float32).at[0, :NC].set(bg)

    for i in range(L):
        e_i = _e_mm(eap, We[i], be[i].reshape(1, H))
        h_in, vt = _pre(h, batchf, vn)
        aggr2 = _edge_call(h_in, e_i, sdm, zeros_buf)
        h = _mlp(aggr2, h_in, W1f[i], b1f[i].reshape(1, 2 * H), W2[i],
                 b2[i].reshape(1, H))
        if i < L - 1:
            vn = _vn_mlp(vt, vn, V1f[i], c1f[i].reshape(1, 2 * H),
                         pa1[i].reshape(1, 1), V2f[i],
                         c2f[i].reshape(1, H), pa2[i].reshape(1, 1))

    logits = _final(h, batchf, wg_pad, bg_pad)
    return logits[:, :NC]


# speedup vs baseline: 1.2212x; 1.2212x over previous
import functools

import jax
import jax.numpy as jnp
from jax import lax
from jax.experimental import pallas as pl
from jax.experimental.pallas import tpu as pltpu
from jax.experimental.pallas import tpu_sc as plsc

N = 10000
E = 320000
H = 128
DE = 16
L = 3
G = 128
NC = 10

NUM_CORES = 2
NUM_SUBCORES = 16
NW = NUM_CORES * NUM_SUBCORES
CHUNK = 128
CH_PER_W = 79
E_PAD = NW * CHUNK * CH_PER_W
NPAD = 10112
ROWS_PER_TILE = NPAD // NUM_SUBCORES

NT = 1000
NGRID = N // NT
ET = 1024
EGRID = E_PAD // ET


def _edge_body(h_hbm, e_hbm, src_hbm, dstm_hbm, zeros_hbm, out_hbm,
               idxs_v, idxd_v, rows_v, e_v, aggr_sh, sem):
    cid = lax.axis_index("c")
    sid = lax.axis_index("s")
    w = cid * NUM_SUBCORES + sid
    r0 = sid * ROWS_PER_TILE
    pltpu.sync_copy(zeros_hbm.at[pl.ds(r0, ROWS_PER_TILE)],
                    aggr_sh.at[pl.ds(r0, ROWS_PER_TILE)])
    plsc.subcore_barrier()

    base = w * CH_PER_W

    def chunk_body(k, carry):
        row = base + k
        pltpu.sync_copy(src_hbm.at[pl.ds(row * CHUNK, CHUNK)], idxs_v)
        pltpu.sync_copy(dstm_hbm.at[pl.ds(row, 1)], idxd_v)
        pltpu.async_copy(h_hbm.at[idxs_v], rows_v, sem).wait()
        pltpu.sync_copy(e_hbm.at[pl.ds(row * CHUNK, CHUNK)], e_v)

        def vbody(i, cc):
            for j in range(H // 16):
                s = (rows_v[i, pl.ds(j * 16, 16)]
                     + e_v[i, pl.ds(j * 16, 16)])
                rows_v[i, pl.ds(j * 16, 16)] = jnp.maximum(s, 0.0)
            return cc

        lax.fori_loop(0, CHUNK, vbody, 0)

        pltpu.sync_copy(rows_v, aggr_sh.at[idxd_v.at[0]], add=True)
        return carry

    lax.fori_loop(0, CH_PER_W, chunk_body, 0)
    plsc.subcore_barrier()
    pltpu.sync_copy(aggr_sh.at[pl.ds(r0, ROWS_PER_TILE)],
                    out_hbm.at[cid, pl.ds(r0, ROWS_PER_TILE)])


_edge_call = functools.partial(
    pl.kernel,
    out_type=jax.ShapeDtypeStruct((NUM_CORES, NPAD, H), jnp.float32),
    mesh=plsc.VectorSubcoreMesh(core_axis_name="c", subcore_axis_name="s",
                                num_cores=NUM_CORES, num_subcores=NUM_SUBCORES),
    scratch_types=[
        pltpu.VMEM((CHUNK,), jnp.int32),
        pltpu.VMEM((1, CHUNK), jnp.int32),
        pltpu.VMEM((CHUNK, H), jnp.float32),
        pltpu.VMEM((CHUNK, H), jnp.float32),
        pltpu.VMEM_SHARED((NPAD, H), jnp.float32),
        pltpu.SemaphoreType.DMA,
    ],
)(_edge_body)


def _e_mm_body(ea_ref, w_ref, b_ref, o_ref):
    o_ref[...] = (jnp.dot(ea_ref[...], w_ref[...],
                          preferred_element_type=jnp.float32) + b_ref[...])


_e_mm = pl.pallas_call(
    _e_mm_body,
    grid=(EGRID,),
    in_specs=[
        pl.BlockSpec((ET, DE), lambda i: (i, 0)),
        pl.BlockSpec((DE, H), lambda i: (0, 0)),
        pl.BlockSpec((1, H), lambda i: (0, 0)),
    ],
    out_specs=pl.BlockSpec((ET, H), lambda i: (i, 0)),
    out_shape=jax.ShapeDtypeStruct((E_PAD, H), jnp.float32),
)


def _pre_body(h_ref, bf_ref, vn_ref, hin_ref, vt_ref):
    i = pl.program_id(0)
    iota = lax.broadcasted_iota(jnp.int32, (NT, G), 1)
    oh = jnp.where(bf_ref[...] == iota, 1.0, 0.0)
    hin = h_ref[...] + jnp.dot(oh, vn_ref[...], preferred_element_type=jnp.float32)
    hin_ref[...] = hin
    contrib = lax.dot_general(oh, hin, (((0,), (0,)), ((), ())),
                              preferred_element_type=jnp.float32)

    @pl.when(i == 0)
    def _():
        vt_ref[...] = jnp.zeros_like(vt_ref)

    vt_ref[...] += contrib


_pre = pl.pallas_call(
    _pre_body,
    grid=(NGRID,),
    in_specs=[
        pl.BlockSpec((NT, H), lambda i: (i, 0)),
        pl.BlockSpec((NT, 1), lambda i: (i, 0)),
        pl.BlockSpec((G, H), lambda i: (0, 0)),
    ],
    out_specs=[
        pl.BlockSpec((NT, H), lambda i: (i, 0)),
        pl.BlockSpec((G, H), lambda i: (0, 0)),
    ],
    out_shape=[
        jax.ShapeDtypeStruct((N, H), jnp.float32),
        jax.ShapeDtypeStruct((G, H), jnp.float32),
    ],
)


def _mlp_body(aggr_ref, hin_ref, w1_ref, b1_ref, w2_ref, b2_ref, o_ref):
    xx = aggr_ref[0] + aggr_ref[1] + hin_ref[...]
    z = jnp.dot(xx, w1_ref[...], preferred_element_type=jnp.float32) + b1_ref[...]
    z = jnp.maximum(z, 0.0)
    o_ref[...] = (jnp.dot(z, w2_ref[...], preferred_element_type=jnp.float32)
                  + b2_ref[...])


_mlp = pl.pallas_call(
    _mlp_body,
    grid=(NGRID,),
    in_specs=[
        pl.BlockSpec((NUM_CORES, NT, H), lambda i: (0, i, 0)),
        pl.BlockSpec((NT, H), lambda i: (i, 0)),
        pl.BlockSpec((H, 2 * H), lambda i: (0, 0)),
        pl.BlockSpec((1, 2 * H), lambda i: (0, 0)),
        pl.BlockSpec((2 * H, H), lambda i: (0, 0)),
        pl.BlockSpec((1, H), lambda i: (0, 0)),
    ],
    out_specs=pl.BlockSpec((NT, H), lambda i: (i, 0)),
    out_shape=jax.ShapeDtypeStruct((N, H), jnp.float32),
)


def _vn_body(vt_ref, vn_ref, v1_ref, c1_ref, a1_ref, v2_ref, c2_ref, a2_ref,
             o_ref):
    u = jnp.dot(vt_ref[...] + vn_ref[...], v1_ref[...],
                preferred_element_type=jnp.float32) + c1_ref[...]
    a1 = a1_ref[0, 0]
    u = jnp.where(u >= 0.0, u, a1 * u)
    u = jnp.dot(u, v2_ref[...], preferred_element_type=jnp.float32) + c2_ref[...]
    a2 = a2_ref[0, 0]
    o_ref[...] = jnp.where(u >= 0.0, u, a2 * u)


_vn_mlp = pl.pallas_call(
    _vn_body,
    out_shape=jax.ShapeDtypeStruct((G, H), jnp.float32),
)


def _final_body(h_ref, bf_ref, wg_ref, bg_ref, o_ref, num_acc, cnt_acc):
    i = pl.program_id(0)
    iota = lax.broadcasted_iota(jnp.int32, (NT, G), 1)
    oh = jnp.where(bf_ref[...] == iota, 1.0, 0.0)

    @pl.when(i == 0)
    def _():
        num_acc[...] = jnp.zeros_like(num_acc)
        cnt_acc[...] = jnp.zeros_like(cnt_acc)

    num_acc[...] += lax.dot_general(oh, h_ref[...], (((0,), (0,)), ((), ())),
                                    preferred_element_type=jnp.float32)
    ones = jnp.ones((NT, H), jnp.float32)
    cnt_acc[...] += lax.dot_general(oh, ones, (((0,), (0,)), ((), ())),
                                    preferred_element_type=jnp.float32)

    @pl.when(i == NGRID - 1)
    def _():
        hg = num_acc[...] / jnp.maximum(cnt_acc[...], 1.0)
        o_ref[...] = (jnp.dot(hg, wg_ref[...],
                              preferred_element_type=jnp.float32) + bg_ref[...])


_final = pl.pallas_call(
    _final_body,
    grid=(NGRID,),
    in_specs=[
        pl.BlockSpec((NT, H), lambda i: (i, 0)),
        pl.BlockSpec((NT, 1), lambda i: (i, 0)),
        pl.BlockSpec((H, H), lambda i: (0, 0)),
        pl.BlockSpec((1, H), lambda i: (0, 0)),
    ],
    out_specs=pl.BlockSpec((G, H), lambda i: (0, 0)),
    out_shape=jax.ShapeDtypeStruct((G, H), jnp.float32),
    scratch_shapes=[
        pltpu.VMEM((G, H), jnp.float32),
        pltpu.VMEM((G, H), jnp.float32),
    ],
)


def kernel(x, edge_index, edge_attr, batch, node_emb, vn_emb, We, be, W1, b1,
           g1, bt1, W2, b2, V1, c1, vg1, vb1, pa1, V2, c2, vg2, vb2, pa2,
           Wg, bg):
    s = 1.0 / jnp.sqrt(jnp.float32(1.0 + 1e-5))
    W1f = W1 * (g1 * s)[:, None, :]
    b1f = b1 * (g1 * s) + bt1
    V1f = V1 * (vg1 * s)[:, None, :]
    c1f = c1 * (vg1 * s) + vb1
    V2f = V2 * (vg2 * s)[:, None, :]
    c2f = c2 * (vg2 * s) + vb2

    h = jnp.take(node_emb, x, axis=0)
    vn = jnp.take(vn_emb, jnp.zeros((G,), jnp.int32), axis=0)
    batchf = batch.reshape(N, 1)

    pad = E_PAD - E
    srcp = jnp.concatenate([edge_index[0], jnp.zeros((pad,), jnp.int32)])
    dstp = jnp.concatenate([edge_index[1], jnp.full((pad,), N, jnp.int32)])
    dstm = dstp.reshape(E_PAD // CHUNK, CHUNK)
    eap = jnp.concatenate([edge_attr, jnp.zeros((pad, DE), jnp.float32)])
    zeros_buf = jnp.zeros((NPAD, H), jnp.float32)

    wg_pad = jnp.zeros((H, H), jnp.float32).at[:, :NC].set(Wg)
    bg_pad = jnp.zeros((1, H), jnp.float32).at[0, :NC].set(bg)

    for i in range(L):
        e_i = _e_mm(eap, We[i], be[i].reshape(1, H))
        h_in, vt = _pre(h, batchf, vn)
        aggr2 = _edge_call(h_in, e_i, srcp, dstm, zeros_buf)
        h = _mlp(aggr2, h_in, W1f[i], b1f[i].reshape(1, 2 * H), W2[i],
                 b2[i].reshape(1, H))
        if i < L - 1:
            vn = _vn_mlp(vt, vn, V1f[i], c1f[i].reshape(1, 2 * H),
                         pa1[i].reshape(1, 1), V2f[i],
                         c2f[i].reshape(1, H), pa2[i].reshape(1, 1))

    logits = _final(h, batchf, wg_pad, bg_pad)
    return logits[:, :NC]
